# baseline (speedup 1.0000x reference)
import functools

import jax
import jax.numpy as jnp
from jax import lax
from jax.experimental import pallas as pl
from jax.experimental.pallas import tpu as pltpu
from jax.experimental.pallas import tpu_sc as plsc

_B, _S, _D = 4, 4096, 1024
_NC, _NS = 2, 16
_NW = _NC * _NS
_ROWS_W = _S // _NW
_CH = 32
_NCHUNK = _ROWS_W // _CH
_CHW = _CH * _D
_NVEC = _CHW // 16


def _sc_add(x_hbm, pos_hbm, out_hbm, pbuf, xbuf, psem, xsem, osem):
    wid = lax.axis_index("s") * _NC + lax.axis_index("c")
    row0 = wid * _ROWS_W

    def add_loop(buf):
        def body(i, _):
            off = i * 16
            buf[pl.ds(off, 16)] = buf[pl.ds(off, 16)] + pbuf[pl.ds(off, 16)]
            return 0
        lax.fori_loop(0, _NVEC, body, 0)

    for c in range(_NCHUNK):
        s0 = row0 + c * _CH
        pltpu.async_copy(pos_hbm.at[pl.ds(s0 * _D, _CHW)], pbuf, psem).wait()
        for b in range(_B):
            base = (b * _S + s0) * _D
            pltpu.async_copy(x_hbm.at[pl.ds(base, _CHW)], xbuf, xsem).wait()
            add_loop(xbuf)
            pltpu.async_copy(xbuf, out_hbm.at[pl.ds(base, _CHW)], osem).wait()


_mesh = plsc.VectorSubcoreMesh(core_axis_name="c", subcore_axis_name="s")

_call = functools.partial(
    pl.kernel,
    out_type=jax.ShapeDtypeStruct((_B * _S * _D,), jnp.float32),
    mesh=_mesh,
    scratch_types=[
        pltpu.VMEM((_CHW,), jnp.float32),
        pltpu.VMEM((_CHW,), jnp.float32),
        pltpu.SemaphoreType.DMA,
        pltpu.SemaphoreType.DMA,
        pltpu.SemaphoreType.DMA,
    ],
)(_sc_add)


@jax.jit
def kernel(x, pos_table):
    x_flat = x.reshape(-1)
    pos_flat = pos_table.reshape(-1)
    out = _call(x_flat, pos_flat)
    return out.reshape(_B, _S, _D)


# speedup vs baseline: 1.4737x; 1.4737x over previous
import functools

import jax
import jax.numpy as jnp
from jax import lax
from jax.experimental import pallas as pl
from jax.experimental.pallas import tpu as pltpu
from jax.experimental.pallas import tpu_sc as plsc

_B, _S, _D = 4, 4096, 1024
_NC, _NS = 2, 16
_NW = _NC * _NS
_ROWS_W = _S // _NW
_CH = 16
_NCHUNK = _ROWS_W // _CH
_CHW = _CH * _D
_NVEC = _CHW // 16
_UNROLL = 8


def _sc_add(x_hbm, pos_hbm, out_hbm, xbuf, pbuf, obuf,
            lsem0, lsem1, ssem0, ssem1, psem0, psem1):
    wid = lax.axis_index("s") * _NC + lax.axis_index("c")
    row0 = wid * _ROWS_W
    lsems = (lsem0, lsem1)
    ssems = (ssem0, ssem1)
    psems = (psem0, psem1)

    def x_off(c, b):
        return (b * _S + row0 + c * _CH) * _D

    def load_x(i):
        c, b = divmod(i, _B)
        return pltpu.async_copy(
            x_hbm.at[pl.ds(x_off(c, b), _CHW)], xbuf.at[i % 2], lsems[i % 2])

    def load_pos(c):
        return pltpu.async_copy(
            pos_hbm.at[pl.ds((row0 + c * _CH) * _D, _CHW)],
            pbuf.at[c % 2], psems[c % 2])

    n_items = _NCHUNK * _B
    load_h = [None] * n_items
    store_h = [None] * n_items
    pos_h = [None] * _NCHUNK

    pos_h[0] = load_pos(0)
    load_h[0] = load_x(0)

    for i in range(n_items):
        c, b = divmod(i, _B)
        if b == 0:
            if c + 1 < _NCHUNK:
                pos_h[c + 1] = load_pos(c + 1)
            pos_h[c].wait()
        if i + 1 < n_items:
            load_h[i + 1] = load_x(i + 1)
        load_h[i].wait()
        if i >= 2:
            store_h[i - 2].wait()

        xb = xbuf.at[i % 2]
        ob = obuf.at[i % 2]
        pb = pbuf.at[c % 2]

        @plsc.parallel_loop(0, _NVEC // _UNROLL)
        def add_body(j):
            base = j * (_UNROLL * 16)
            for k in range(_UNROLL):
                off = base + k * 16
                ob[pl.ds(off, 16)] = xb[pl.ds(off, 16)] + pb[pl.ds(off, 16)]

        store_h[i] = pltpu.async_copy(
            obuf.at[i % 2], out_hbm.at[pl.ds(x_off(c, b), _CHW)], ssems[i % 2])

    store_h[n_items - 2].wait()
    store_h[n_items - 1].wait()


_mesh = plsc.VectorSubcoreMesh(core_axis_name="c", subcore_axis_name="s")

_call = functools.partial(
    pl.kernel,
    out_type=jax.ShapeDtypeStruct((_B * _S * _D,), jnp.float32),
    mesh=_mesh,
    scratch_types=[
        pltpu.VMEM((2, _CHW), jnp.float32),
        pltpu.VMEM((2, _CHW), jnp.float32),
        pltpu.VMEM((2, _CHW), jnp.float32),
        pltpu.SemaphoreType.DMA,
        pltpu.SemaphoreType.DMA,
        pltpu.SemaphoreType.DMA,
        pltpu.SemaphoreType.DMA,
        pltpu.SemaphoreType.DMA,
        pltpu.SemaphoreType.DMA,
    ],
)(_sc_add)


@jax.jit
def kernel(x, pos_table):
    x_flat = x.reshape(-1)
    pos_flat = pos_table.reshape(-1)
    out = _call(x_flat, pos_flat)
    return out.reshape(_B, _S, _D)


# speedup vs baseline: 4.7447x; 3.2197x over previous
import functools

import jax
import jax.numpy as jnp
from jax import lax
from jax.experimental import pallas as pl
from jax.experimental.pallas import tpu as pltpu
from jax.experimental.pallas import tpu_sc as plsc

_B, _S, _D = 4, 4096, 1024
_NC, _NS = 2, 16
_NW = _NC * _NS
_ROWS_W = _S // _NW
_CH = 16
_NCHUNK = _ROWS_W // _CH
_NVEC = _CH * _D // 16
_CPR = _D // 16


def _sc_add(x_hbm, pos_hbm, out_hbm, xbuf, pbuf, obuf,
            lsem0, lsem1, ssem0, ssem1, psem0, psem1):
    wid = lax.axis_index("s") * _NC + lax.axis_index("c")
    row0 = wid * _ROWS_W
    lsems = (lsem0, lsem1)
    ssems = (ssem0, ssem1)
    psems = (psem0, psem1)

    def load_x(i):
        c, b = divmod(i, _B)
        return pltpu.async_copy(
            x_hbm.at[b, pl.ds(row0 + c * _CH, _CH)], xbuf.at[i % 2],
            lsems[i % 2])

    def load_pos(c):
        return pltpu.async_copy(
            pos_hbm.at[pl.ds(row0 + c * _CH, _CH)], pbuf.at[c % 2],
            psems[c % 2])

    n_items = _NCHUNK * _B
    load_h = [None] * n_items
    store_h = [None] * n_items
    pos_h = [None] * _NCHUNK

    pos_h[0] = load_pos(0)
    load_h[0] = load_x(0)

    for i in range(n_items):
        c, b = divmod(i, _B)
        if b == 0:
            if c + 1 < _NCHUNK:
                pos_h[c + 1] = load_pos(c + 1)
            pos_h[c].wait()
        if i + 1 < n_items:
            load_h[i + 1] = load_x(i + 1)
        load_h[i].wait()
        if i >= 2:
            store_h[i - 2].wait()

        xb = xbuf.at[i % 2]
        ob = obuf.at[i % 2]
        pb = pbuf.at[c % 2]

        @plsc.parallel_loop(0, _NVEC, unroll=8)
        def add_body(j):
            r = j >> 6
            cc = (j & (_CPR - 1)) * 16
            ob[r, pl.ds(cc, 16)] = xb[r, pl.ds(cc, 16)] + pb[r, pl.ds(cc, 16)]

        store_h[i] = pltpu.async_copy(
            obuf.at[i % 2], out_hbm.at[b, pl.ds(row0 + c * _CH, _CH)],
            ssems[i % 2])

    store_h[n_items - 2].wait()
    store_h[n_items - 1].wait()


_mesh = plsc.VectorSubcoreMesh(core_axis_name="c", subcore_axis_name="s")

_call = functools.partial(
    pl.kernel,
    out_type=jax.ShapeDtypeStruct((_B, _S, _D), jnp.float32),
    mesh=_mesh,
    scratch_types=[
        pltpu.VMEM((2, _CH, _D), jnp.float32),
        pltpu.VMEM((2, _CH, _D), jnp.float32),
        pltpu.VMEM((2, _CH, _D), jnp.float32),
        pltpu.SemaphoreType.DMA,
        pltpu.SemaphoreType.DMA,
        pltpu.SemaphoreType.DMA,
        pltpu.SemaphoreType.DMA,
        pltpu.SemaphoreType.DMA,
        pltpu.SemaphoreType.DMA,
    ],
)(_sc_add)


@jax.jit
def kernel(x, pos_table):
    return _call(x, pos_table)
